# baseline (speedup 1.0000x reference)
import jax
import jax.numpy as jnp
from jax.experimental import pallas as pl
from jax.experimental.pallas import tpu as pltpu

_RBF_START = 0.0
_RBF_END = 8.0
_RBF_STEPS = 16
_ENERGY_SCALE = 0.01
_EPS = 1e-10

_L = 1024
_R = 4096
_BR = 512

_LOG2E = 1.4426950408889634


def _ff_body(lf_ref, rf_ref, lc_ref, rc_ref, out_ref):
    d2 = jnp.zeros((_L, _BR), jnp.float32)
    for axis in range(3):
        diff = lc_ref[:, axis][:, None] - rc_ref[axis, :][None, :]
        d2 = d2 + (diff * diff + _EPS)

    c = 2.0 * (_LOG2E ** 0.5)
    dc = jnp.sqrt(d2 * (c * c))
    ndc = -dc

    acc = jnp.zeros((_L, _BR), jnp.float32)
    for e in range(_RBF_STEPS):
        mu_e = _RBF_START + e * (_RBF_END - _RBF_START) / (_RBF_STEPS - 1)
        w = dc - (mu_e * c)
        nw = ndc + (mu_e * c)
        rbf = jnp.exp2(w * nw)
        atn = jax.lax.dot(lf_ref[e], rf_ref[e],
                          preferred_element_type=jnp.float32)
        acc = acc + rbf * atn
    s = jnp.sum(acc) * _ENERGY_SCALE
    out_ref[...] = jnp.full((1, 1, 128), s, jnp.float32)


def kernel(lig_feat, rec_feat, lig_coord, rec_coord, weight, bias):
    lf_t = jnp.transpose(lig_feat, (1, 0, 2))
    rf_t = jnp.transpose(rec_feat, (1, 2, 0))
    rc_t = jnp.transpose(rec_coord, (1, 0))

    grid = (_R // _BR,)
    partials = pl.pallas_call(
        _ff_body,
        grid=grid,
        in_specs=[
            pl.BlockSpec((_RBF_STEPS, _L, 16), lambda j: (0, 0, 0)),
            pl.BlockSpec((_RBF_STEPS, 16, _BR), lambda j: (0, 0, j)),
            pl.BlockSpec((_L, 3), lambda j: (0, 0)),
            pl.BlockSpec((3, _BR), lambda j: (0, j)),
        ],
        out_specs=pl.BlockSpec((1, 1, 128), lambda j: (j, 0, 0)),
        out_shape=jax.ShapeDtypeStruct((_R // _BR, 1, 128), jnp.float32),
        compiler_params=pltpu.CompilerParams(
            dimension_semantics=("arbitrary",),
        ),
    )(lf_t, rf_t, lig_coord, rc_t)

    u = jnp.sum(partials[:, 0, 0])
    return bias.reshape(()) + u * weight.reshape(())


# speedup vs baseline: 1.0766x; 1.0766x over previous
import jax
import jax.numpy as jnp
from jax.experimental import pallas as pl
from jax.experimental.pallas import tpu as pltpu

_RBF_START = 0.0
_RBF_END = 8.0
_RBF_STEPS = 16
_ENERGY_SCALE = 0.01
_EPS = 1e-10

_L = 1024
_R = 4096
_BR = 1024

_LOG2E = 1.4426950408889634


def _ff_body(lf_ref, rf_ref, lc_ref, rc_ref, out_ref):
    d2 = jnp.zeros((_L, _BR), jnp.float32)
    for axis in range(3):
        diff = lc_ref[:, axis][:, None] - rc_ref[axis, :][None, :]
        d2 = d2 + (diff * diff + _EPS)

    c = 2.0 * (_LOG2E ** 0.5)
    dc = jnp.sqrt(d2 * (c * c))

    s = jnp.float32(0.0)
    for e in range(_RBF_STEPS):
        mu_e = _RBF_START + e * (_RBF_END - _RBF_START) / (_RBF_STEPS - 1)
        w = dc - (mu_e * c)
        rbf = jnp.exp2(-(w * w))
        atn = jax.lax.dot(lf_ref[e], rf_ref[e],
                          preferred_element_type=jnp.float32)
        s = s + jnp.sum(rbf * atn)
    s = s * _ENERGY_SCALE
    out_ref[...] = jnp.full((1, 1, 128), s, jnp.float32)


def kernel(lig_feat, rec_feat, lig_coord, rec_coord, weight, bias):
    lf_t = jnp.transpose(lig_feat, (1, 0, 2))
    rf_t = jnp.transpose(rec_feat, (1, 2, 0))
    rc_t = jnp.transpose(rec_coord, (1, 0))

    grid = (_R // _BR,)
    partials = pl.pallas_call(
        _ff_body,
        grid=grid,
        in_specs=[
            pl.BlockSpec((_RBF_STEPS, _L, 16), lambda j: (0, 0, 0)),
            pl.BlockSpec((_RBF_STEPS, 16, _BR), lambda j: (0, 0, j)),
            pl.BlockSpec((_L, 3), lambda j: (0, 0)),
            pl.BlockSpec((3, _BR), lambda j: (0, j)),
        ],
        out_specs=pl.BlockSpec((1, 1, 128), lambda j: (j, 0, 0)),
        out_shape=jax.ShapeDtypeStruct((_R // _BR, 1, 128), jnp.float32),
        compiler_params=pltpu.CompilerParams(
            dimension_semantics=("arbitrary",),
        ),
    )(lf_t, rf_t, lig_coord, rc_t)

    u = jnp.sum(partials[:, 0, 0])
    return bias.reshape(()) + u * weight.reshape(())
